# Initial kernel scaffold; baseline (speedup 1.0000x reference)
import jax
import jax.numpy as jnp
from jax.experimental import pallas as pl


def kernel(z, edge_distance, edge_index, q, p, covalent_radii):
    raise NotImplementedError("write your pallas kernel here")


# speedup vs baseline: 757.5972x; 757.5972x over previous
import dataclasses
import functools

import jax
import jax.numpy as jnp
from jax import lax
from jax.experimental import pallas as pl
from jax.experimental.pallas import tpu as pltpu
from jax.experimental.pallas import tpu_sc as plsc

_N_TILES = 32
_LANES = 16


@functools.lru_cache(maxsize=None)
def _build_sc_r0(n_edges: int, n_nodes: int, rad_len: int):
    edges_per_tile = n_edges // _N_TILES
    chunk = 8000
    n_chunks = edges_per_tile // chunk
    assert edges_per_tile % chunk == 0 and chunk % _LANES == 0

    def body(z_hbm, send_hbm, recv_hbm, rad_hbm, out_hbm, z_v, rad_v, s_v, r_v, o_v):
        wid = lax.axis_index("s") * 2 + lax.axis_index("c")
        pltpu.sync_copy(z_hbm, z_v)
        pltpu.sync_copy(rad_hbm, rad_v)
        base = wid * edges_per_tile

        @pl.loop(0, n_chunks)
        def _chunk(j):
            eb = base + j * chunk
            pltpu.sync_copy(send_hbm.at[pl.ds(eb, chunk)], s_v)
            pltpu.sync_copy(recv_hbm.at[pl.ds(eb, chunk)], r_v)

            @pl.loop(0, chunk, step=_LANES)
            def _vec(i):
                sv = s_v[pl.ds(i, _LANES)]
                rv = r_v[pl.ds(i, _LANES)]
                zs = plsc.load_gather(z_v, [sv])
                zr = plsc.load_gather(z_v, [rv])
                rs = plsc.load_gather(rad_v, [zs])
                rr = plsc.load_gather(rad_v, [zr])
                o_v[pl.ds(i, _LANES)] = rs + rr

            pltpu.sync_copy(o_v, out_hbm.at[pl.ds(eb, chunk)])

    cp = pltpu.CompilerParams()
    if "needs_layout_passes" in pltpu.CompilerParams.__dataclass_fields__:
        cp = dataclasses.replace(cp, needs_layout_passes=False)
    return pl.kernel(
        body,
        out_type=jax.ShapeDtypeStruct((n_edges,), jnp.float32),
        compiler_params=cp,
        mesh=plsc.VectorSubcoreMesh(
            core_axis_name="c", subcore_axis_name="s",
            num_cores=2, num_subcores=16,
        ),
        scratch_types=[
            pltpu.VMEM((n_nodes,), jnp.int32),
            pltpu.VMEM((rad_len,), jnp.float32),
            pltpu.VMEM((chunk,), jnp.int32),
            pltpu.VMEM((chunk,), jnp.int32),
            pltpu.VMEM((chunk,), jnp.float32),
        ],
    )


def _tc_w_body(s_ref, d_ref, r0_ref, o_ref):
    t = d_ref[...] / r0_ref[...]
    lt = jnp.log(t)
    tq = jnp.exp(s_ref[1] * lt)
    tqp = jnp.exp(s_ref[2] * lt)
    atq = s_ref[0] * tq
    o_ref[...] = atq / (1.0 + tqp + atq)


@functools.lru_cache(maxsize=None)
def _build_tc_w(n_edges: int):
    cols = 128
    rows = n_edges // cols
    block_rows = 2000
    assert rows % block_rows == 0
    grid = rows // block_rows
    return pl.pallas_call(
        _tc_w_body,
        out_shape=jax.ShapeDtypeStruct((rows, cols), jnp.float32),
        grid=(grid,),
        in_specs=[
            pl.BlockSpec(memory_space=pltpu.SMEM),
            pl.BlockSpec((block_rows, cols), lambda i: (i, 0)),
            pl.BlockSpec((block_rows, cols), lambda i: (i, 0)),
        ],
        out_specs=pl.BlockSpec((block_rows, cols), lambda i: (i, 0)),
    )


def kernel(z, edge_distance, edge_index, q, p, covalent_radii):
    n_edges = edge_distance.shape[0]
    n_nodes = z.shape[0]
    pp = 2.0 * jax.nn.softplus(0.5 * p) + 1.0
    qq = 2.0 * jax.nn.softplus(0.5 * q) + 1.0
    a = -2.0 * (pp + qq - 2.0 * qq * pp) / (pp**2 + pp + qq**2 + qq)
    scalars = jnp.stack([a, qq, qq - pp]).astype(jnp.float32)

    rad_len = 128
    rad = jnp.zeros((rad_len,), jnp.float32).at[: covalent_radii.shape[0]].set(
        covalent_radii
    )
    r0 = _build_sc_r0(n_edges, n_nodes, rad_len)(
        z, edge_index[0], edge_index[1], rad
    )

    cols = 128
    d2 = edge_distance.reshape(n_edges // cols, cols)
    r02 = r0.reshape(n_edges // cols, cols)
    w = _build_tc_w(n_edges)(scalars, d2, r02)
    return w.reshape(n_edges)


# speedup vs baseline: 933.8880x; 1.2327x over previous
import dataclasses
import functools

import jax
import jax.numpy as jnp
from jax import lax
from jax.experimental import pallas as pl
from jax.experimental.pallas import tpu as pltpu
from jax.experimental.pallas import tpu_sc as plsc

_N_TILES = 32
_LANES = 16


@functools.lru_cache(maxsize=None)
def _build_sc_r0(n_edges: int, n_nodes: int, rad_len: int):
    edges_per_tile = n_edges // _N_TILES
    chunk = 4000
    n_chunks = edges_per_tile // chunk
    assert edges_per_tile % chunk == 0 and chunk % _LANES == 0
    assert n_chunks % 2 == 0 and n_chunks >= 4

    def body(eif_hbm, z_hbm, rad_hbm, out_hbm,
             zrn_v, rad_v, s0, s1, r0_, r1_, o0, o1,
             ss0, ss1, sr0, sr1, so0, so1):
        wid = lax.axis_index("s") * 2 + lax.axis_index("c")
        base = wid * edges_per_tile
        s_bufs, r_bufs, o_bufs = (s0, s1), (r0_, r1_), (o0, o1)
        sem_s, sem_r, sem_o = (ss0, ss1), (sr0, sr1), (so0, so1)

        pltpu.sync_copy(z_hbm, zrn_v)
        pltpu.sync_copy(rad_hbm, rad_v)

        @pl.loop(0, n_nodes, step=_LANES, unroll=8)
        def _rn(i):
            zv = zrn_v[pl.ds(i, _LANES)]
            rv = plsc.load_gather(rad_v, [zv])
            zrn_v[pl.ds(i, _LANES)] = plsc.bitcast(rv, jnp.int32)

        def start_in(jj, b):
            eb = base + jj * chunk
            pltpu.async_copy(eif_hbm.at[pl.ds(eb, chunk)], s_bufs[b], sem_s[b])
            pltpu.async_copy(
                eif_hbm.at[pl.ds(n_edges + eb, chunk)], r_bufs[b], sem_r[b]
            )

        def wait_in(b):
            pltpu.make_async_copy(
                eif_hbm.at[pl.ds(0, chunk)], s_bufs[b], sem_s[b]
            ).wait()
            pltpu.make_async_copy(
                eif_hbm.at[pl.ds(0, chunk)], r_bufs[b], sem_r[b]
            ).wait()

        def start_out(jj, b):
            eb = base + jj * chunk
            pltpu.async_copy(o_bufs[b], out_hbm.at[pl.ds(eb, chunk)], sem_o[b])

        def wait_out(b):
            pltpu.make_async_copy(
                o_bufs[b], out_hbm.at[pl.ds(0, chunk)], sem_o[b]
            ).wait()

        start_in(0, 0)
        start_in(1, 1)

        @pl.loop(0, n_chunks, step=2)
        def _chunks(j):
            for b in range(2):
                jj = j + b

                @pl.when(jj >= 2)
                def _():
                    wait_out(b)

                wait_in(b)

                @pl.loop(0, chunk, step=_LANES, unroll=8)
                def _vec(i):
                    sv = s_bufs[b][pl.ds(i, _LANES)]
                    rv = r_bufs[b][pl.ds(i, _LANES)]
                    rs = plsc.bitcast(plsc.load_gather(zrn_v, [sv]), jnp.float32)
                    rr = plsc.bitcast(plsc.load_gather(zrn_v, [rv]), jnp.float32)
                    o_bufs[b][pl.ds(i, _LANES)] = rs + rr

                start_out(jj, b)

                @pl.when(jj + 2 < n_chunks)
                def _():
                    start_in(jj + 2, b)

        wait_out(0)
        wait_out(1)

    cp = pltpu.CompilerParams()
    if "needs_layout_passes" in pltpu.CompilerParams.__dataclass_fields__:
        cp = dataclasses.replace(cp, needs_layout_passes=False)
    return pl.kernel(
        body,
        out_type=jax.ShapeDtypeStruct((n_edges,), jnp.float32),
        compiler_params=cp,
        mesh=plsc.VectorSubcoreMesh(
            core_axis_name="c", subcore_axis_name="s",
            num_cores=2, num_subcores=16,
        ),
        scratch_types=[
            pltpu.VMEM((n_nodes,), jnp.int32),
            pltpu.VMEM((rad_len,), jnp.float32),
            pltpu.VMEM((chunk,), jnp.int32),
            pltpu.VMEM((chunk,), jnp.int32),
            pltpu.VMEM((chunk,), jnp.int32),
            pltpu.VMEM((chunk,), jnp.int32),
            pltpu.VMEM((chunk,), jnp.float32),
            pltpu.VMEM((chunk,), jnp.float32),
            pltpu.SemaphoreType.DMA,
            pltpu.SemaphoreType.DMA,
            pltpu.SemaphoreType.DMA,
            pltpu.SemaphoreType.DMA,
            pltpu.SemaphoreType.DMA,
            pltpu.SemaphoreType.DMA,
        ],
    )


def _tc_w_body(s_ref, d_ref, r0_ref, o_ref):
    t = d_ref[...] / r0_ref[...]
    lt = jnp.log(t)
    tq = jnp.exp(s_ref[1] * lt)
    tqp = jnp.exp(s_ref[2] * lt)
    atq = s_ref[0] * tq
    o_ref[...] = atq / (1.0 + tqp + atq)


@functools.lru_cache(maxsize=None)
def _build_tc_w(n_edges: int):
    cols = 128
    rows = n_edges // cols
    block_rows = 2000
    assert rows % block_rows == 0
    grid = rows // block_rows
    return pl.pallas_call(
        _tc_w_body,
        out_shape=jax.ShapeDtypeStruct((rows, cols), jnp.float32),
        grid=(grid,),
        in_specs=[
            pl.BlockSpec(memory_space=pltpu.SMEM),
            pl.BlockSpec((block_rows, cols), lambda i: (i, 0)),
            pl.BlockSpec((block_rows, cols), lambda i: (i, 0)),
        ],
        out_specs=pl.BlockSpec((block_rows, cols), lambda i: (i, 0)),
    )


def kernel(z, edge_distance, edge_index, q, p, covalent_radii):
    n_edges = edge_distance.shape[0]
    n_nodes = z.shape[0]
    pp = 2.0 * jax.nn.softplus(0.5 * p) + 1.0
    qq = 2.0 * jax.nn.softplus(0.5 * q) + 1.0
    a = -2.0 * (pp + qq - 2.0 * qq * pp) / (pp**2 + pp + qq**2 + qq)
    scalars = jnp.stack([a, qq, qq - pp]).astype(jnp.float32)

    rad_len = 128
    rad = jnp.concatenate(
        [covalent_radii.astype(jnp.float32),
         jnp.zeros((rad_len - covalent_radii.shape[0],), jnp.float32)]
    )
    eif = edge_index.reshape(2 * n_edges)
    r0 = _build_sc_r0(n_edges, n_nodes, rad_len)(eif, z, rad)

    cols = 128
    d2 = edge_distance.reshape(n_edges // cols, cols)
    r02 = r0.reshape(n_edges // cols, cols)
    w = _build_tc_w(n_edges)(scalars, d2, r02)
    return w.reshape(n_edges)


# speedup vs baseline: 1655.7936x; 1.7730x over previous
import dataclasses
import functools

import jax
import jax.numpy as jnp
from jax import lax
from jax.experimental import pallas as pl
from jax.experimental.pallas import tpu as pltpu
from jax.experimental.pallas import tpu_sc as plsc

_N_TILES = 32
_LANES = 16


@functools.lru_cache(maxsize=None)
def _build_sc_r0(n_edges: int, n_nodes: int, rad_len: int):
    edges_per_tile = n_edges // _N_TILES
    chunk = 4000
    n_chunks = edges_per_tile // chunk
    assert edges_per_tile % chunk == 0 and chunk % _LANES == 0
    assert n_chunks % 2 == 0 and n_chunks >= 4

    def body(ei_hbm, z_hbm, rad_hbm, out_hbm,
             zrn_v, rad_v, s0, s1, r0_, r1_, o0, o1,
             zsem, ss0, ss1, sr0, sr1, so0, so1):
        wid = lax.axis_index("s") * 2 + lax.axis_index("c")
        base = wid * edges_per_tile
        s_bufs, r_bufs, o_bufs = (s0, s1), (r0_, r1_), (o0, o1)
        sem_s, sem_r, sem_o = (ss0, ss1), (sr0, sr1), (so0, so1)

        def start_in(jj, b):
            eb = base + jj * chunk
            pltpu.async_copy(ei_hbm.at[0, pl.ds(eb, chunk)], s_bufs[b], sem_s[b])
            pltpu.async_copy(ei_hbm.at[1, pl.ds(eb, chunk)], r_bufs[b], sem_r[b])

        def wait_in(b):
            pltpu.make_async_copy(
                ei_hbm.at[0, pl.ds(0, chunk)], s_bufs[b], sem_s[b]
            ).wait()
            pltpu.make_async_copy(
                ei_hbm.at[0, pl.ds(0, chunk)], r_bufs[b], sem_r[b]
            ).wait()

        zcopy = pltpu.async_copy(z_hbm, zrn_v, zsem)
        pltpu.sync_copy(rad_hbm, rad_v)
        start_in(0, 0)
        start_in(1, 1)
        zcopy.wait()

        @plsc.parallel_loop(0, n_nodes, _LANES, unroll=8)
        def _rn(i):
            zv = zrn_v[pl.ds(i, _LANES)]
            rv = plsc.load_gather(rad_v, [zv])
            zrn_v[pl.ds(i, _LANES)] = plsc.bitcast(rv, jnp.int32)

        def start_out(jj, b):
            eb = base + jj * chunk
            pltpu.async_copy(o_bufs[b], out_hbm.at[pl.ds(eb, chunk)], sem_o[b])

        def wait_out(b):
            pltpu.make_async_copy(
                o_bufs[b], out_hbm.at[pl.ds(0, chunk)], sem_o[b]
            ).wait()

        start_in(0, 0)
        start_in(1, 1)

        @pl.loop(0, n_chunks, step=2)
        def _chunks(j):
            for b in range(2):
                jj = j + b

                @pl.when(jj >= 2)
                def _():
                    wait_out(b)

                wait_in(b)

                @plsc.parallel_loop(0, chunk, _LANES, unroll=8)
                def _vec(i):
                    sv = s_bufs[b][pl.ds(i, _LANES)]
                    rv = r_bufs[b][pl.ds(i, _LANES)]
                    rs = plsc.bitcast(plsc.load_gather(zrn_v, [sv]), jnp.float32)
                    rr = plsc.bitcast(plsc.load_gather(zrn_v, [rv]), jnp.float32)
                    o_bufs[b][pl.ds(i, _LANES)] = rs + rr

                start_out(jj, b)

                @pl.when(jj + 2 < n_chunks)
                def _():
                    start_in(jj + 2, b)

        wait_out(0)
        wait_out(1)

    cp = pltpu.CompilerParams(
        needs_layout_passes=False, use_tc_tiling_on_sc=False
    )
    return pl.kernel(
        body,
        out_type=jax.ShapeDtypeStruct((n_edges,), jnp.float32),
        compiler_params=cp,
        mesh=plsc.VectorSubcoreMesh(
            core_axis_name="c", subcore_axis_name="s",
            num_cores=2, num_subcores=16,
        ),
        scratch_types=[
            pltpu.VMEM((n_nodes,), jnp.int32),
            pltpu.VMEM((rad_len,), jnp.float32),
            pltpu.VMEM((chunk,), jnp.int32),
            pltpu.VMEM((chunk,), jnp.int32),
            pltpu.VMEM((chunk,), jnp.int32),
            pltpu.VMEM((chunk,), jnp.int32),
            pltpu.VMEM((chunk,), jnp.float32),
            pltpu.VMEM((chunk,), jnp.float32),
            pltpu.SemaphoreType.DMA,
            pltpu.SemaphoreType.DMA,
            pltpu.SemaphoreType.DMA,
            pltpu.SemaphoreType.DMA,
            pltpu.SemaphoreType.DMA,
            pltpu.SemaphoreType.DMA,
            pltpu.SemaphoreType.DMA,
        ],
    )


def _tc_w_body(s_ref, d_ref, r0_ref, o_ref):
    t = d_ref[...] / r0_ref[...]
    lt = jnp.log(t)
    tq = jnp.exp(s_ref[1] * lt)
    tqp = jnp.exp(s_ref[2] * lt)
    atq = s_ref[0] * tq
    o_ref[...] = atq / (1.0 + tqp + atq)


@functools.lru_cache(maxsize=None)
def _build_tc_w(n_edges: int):
    cols = 128
    rows = n_edges // cols
    block_rows = 2000
    assert rows % block_rows == 0
    grid = rows // block_rows
    return pl.pallas_call(
        _tc_w_body,
        out_shape=jax.ShapeDtypeStruct((rows, cols), jnp.float32),
        grid=(grid,),
        in_specs=[
            pl.BlockSpec(memory_space=pltpu.SMEM),
            pl.BlockSpec((block_rows, cols), lambda i: (i, 0)),
            pl.BlockSpec((block_rows, cols), lambda i: (i, 0)),
        ],
        out_specs=pl.BlockSpec((block_rows, cols), lambda i: (i, 0)),
    )


def kernel(z, edge_distance, edge_index, q, p, covalent_radii):
    n_edges = edge_distance.shape[0]
    n_nodes = z.shape[0]
    pp = 2.0 * jax.nn.softplus(0.5 * p) + 1.0
    qq = 2.0 * jax.nn.softplus(0.5 * q) + 1.0
    a = -2.0 * (pp + qq - 2.0 * qq * pp) / (pp**2 + pp + qq**2 + qq)
    scalars = jnp.stack([a, qq, qq - pp]).astype(jnp.float32)

    rad_len = 128
    rad = jnp.concatenate(
        [covalent_radii.astype(jnp.float32),
         jnp.zeros((rad_len - covalent_radii.shape[0],), jnp.float32)]
    )
    r0 = _build_sc_r0(n_edges, n_nodes, rad_len)(edge_index, z, rad)

    cols = 128
    d2 = edge_distance.reshape(n_edges // cols, cols)
    r02 = r0.reshape(n_edges // cols, cols)
    w = _build_tc_w(n_edges)(scalars, d2, r02)
    return w.reshape(n_edges)
